# baseline (device time: 10948 ns/iter reference)
import jax
import jax.numpy as jnp
from jax import lax
from jax.experimental import pallas as pl
from jax.experimental.pallas import tpu as pltpu

N_DEV = 8
N_TOK = 256
D_IN = 128
D_OUT = 256
N_EXP = 16
EXP_PER_DEV = 2
ROWS = N_TOK // N_DEV


def kernel(x, router_W, route_idx, expert_W):
    def body(x_ref, rw_ref, idx_ref, ew_ref, out_ref,
             partial_ref, acc_ref, send_sems, recv_sems):
        my = lax.axis_index("i")

        bar = pltpu.get_barrier_semaphore()
        for p in range(N_DEV):
            @pl.when(my != p)
            def _():
                pl.semaphore_signal(
                    bar, inc=1, device_id=(p,),
                    device_id_type=pl.DeviceIdType.MESH,
                )
        pl.semaphore_wait(bar, N_DEV - 1)

        xf = x_ref[:, :]
        scores = jnp.dot(xf, rw_ref[:, :], preferred_element_type=jnp.float32)
        smax = jnp.max(scores, axis=1, keepdims=True)
        es = jnp.exp(scores - smax)
        eidx = lax.broadcasted_iota(jnp.int32, (N_TOK, N_EXP), 1)
        i0 = idx_ref[:, 0:1]
        i1 = idx_ref[:, 1:2]
        p0 = jnp.sum(jnp.where(eidx == i0, es, 0.0), axis=1, keepdims=True)
        p1 = jnp.sum(jnp.where(eidx == i1, es, 0.0), axis=1, keepdims=True)
        gs = p0 + p1

        xb = xf.astype(jnp.bfloat16)
        partial = jnp.zeros((N_TOK, D_OUT), jnp.float32)
        for le in range(EXP_PER_DEV):
            eg = my * EXP_PER_DEV + le
            w = (jnp.where(i0 == eg, p0, 0.0)
                 + jnp.where(i1 == eg, p1, 0.0)) / gs
            y = jnp.dot(xb, ew_ref[le].astype(jnp.bfloat16),
                        preferred_element_type=jnp.float32)
            partial = partial + w * y
        partial_ref[:, :] = partial

        rdmas = []
        for k in range(1, N_DEV):
            t = lax.rem(my + k, N_DEV)
            rdma = pltpu.make_async_remote_copy(
                src_ref=partial_ref.at[pl.ds(t * ROWS, ROWS)],
                dst_ref=acc_ref.at[k],
                send_sem=send_sems.at[k],
                recv_sem=recv_sems.at[k],
                device_id=(t,),
                device_id_type=pl.DeviceIdType.MESH,
            )
            rdma.start()
            rdmas.append(rdma)

        out = partial_ref[pl.ds(my * ROWS, ROWS), :]
        for k, rdma in zip(range(1, N_DEV), rdmas):
            rdma.wait()
            out = out + acc_ref[k]
        out_ref[:, :] = out

    return pl.pallas_call(
        body,
        out_shape=jax.ShapeDtypeStruct((ROWS, D_OUT), jnp.float32),
        in_specs=[pl.BlockSpec(memory_space=pltpu.VMEM)] * 4,
        out_specs=pl.BlockSpec(memory_space=pltpu.VMEM),
        scratch_shapes=[
            pltpu.VMEM((N_TOK, D_OUT), jnp.float32),
            pltpu.VMEM((N_DEV, ROWS, D_OUT), jnp.float32),
            pltpu.SemaphoreType.DMA((N_DEV,)),
            pltpu.SemaphoreType.DMA((N_DEV,)),
        ],
        compiler_params=pltpu.CompilerParams(collective_id=0),
    )(x, router_W, route_idx, expert_W)


# device time: 9651 ns/iter; 1.1344x vs baseline; 1.1344x over previous
import jax
import jax.numpy as jnp
from jax import lax
from jax.experimental import pallas as pl
from jax.experimental.pallas import tpu as pltpu

N_DEV = 8
N_TOK = 256
D_IN = 128
D_OUT = 256
N_EXP = 16
EXP_PER_DEV = 2
ROWS = N_TOK // N_DEV


def kernel(x, router_W, route_idx, expert_W):
    def body(x_ref, rw_ref, idx_ref, ew_ref, out_ref,
             partial_ref, acc_ref, send_sems, recv_sems):
        my = lax.axis_index("i")

        bar = pltpu.get_barrier_semaphore()
        for p in range(N_DEV):
            @pl.when(my != p)
            def _():
                pl.semaphore_signal(
                    bar, inc=1, device_id=(p,),
                    device_id_type=pl.DeviceIdType.MESH,
                )

        xf = x_ref[:, :]
        scores = jnp.dot(xf, rw_ref[:, :], preferred_element_type=jnp.float32)
        smax = jnp.max(scores, axis=1, keepdims=True)
        es = jnp.exp(scores - smax)
        eidx = lax.broadcasted_iota(jnp.int32, (N_TOK, N_EXP), 1)
        i0 = idx_ref[:, 0:1]
        i1 = idx_ref[:, 1:2]
        p0 = jnp.sum(jnp.where(eidx == i0, es, 0.0), axis=1, keepdims=True)
        p1 = jnp.sum(jnp.where(eidx == i1, es, 0.0), axis=1, keepdims=True)
        gs = p0 + p1

        xb = xf.astype(jnp.bfloat16)
        partial = jnp.zeros((N_TOK, D_OUT), jnp.float32)
        for le in range(EXP_PER_DEV):
            eg = my * EXP_PER_DEV + le
            w = (jnp.where(i0 == eg, p0, 0.0)
                 + jnp.where(i1 == eg, p1, 0.0)) / gs
            y = jnp.dot(xb, ew_ref[le].astype(jnp.bfloat16),
                        preferred_element_type=jnp.float32)
            partial = partial + w * y
        partial_ref[:, :] = partial.astype(jnp.bfloat16)

        pl.semaphore_wait(bar, N_DEV - 1)

        rdmas = []
        for k in range(1, N_DEV):
            t = lax.rem(my + k, N_DEV)
            rdma = pltpu.make_async_remote_copy(
                src_ref=partial_ref.at[pl.ds(t * ROWS, ROWS)],
                dst_ref=acc_ref.at[k],
                send_sem=send_sems.at[k],
                recv_sem=recv_sems.at[k],
                device_id=(t,),
                device_id_type=pl.DeviceIdType.MESH,
            )
            rdma.start()
            rdmas.append(rdma)

        out = partial_ref[pl.ds(my * ROWS, ROWS), :].astype(jnp.float32)
        for k, rdma in zip(range(1, N_DEV), rdmas):
            rdma.wait()
            out = out + acc_ref[k].astype(jnp.float32)
        out_ref[:, :] = out

    return pl.pallas_call(
        body,
        out_shape=jax.ShapeDtypeStruct((ROWS, D_OUT), jnp.float32),
        in_specs=[pl.BlockSpec(memory_space=pltpu.VMEM)] * 4,
        out_specs=pl.BlockSpec(memory_space=pltpu.VMEM),
        scratch_shapes=[
            pltpu.VMEM((N_TOK, D_OUT), jnp.bfloat16),
            pltpu.VMEM((N_DEV, ROWS, D_OUT), jnp.bfloat16),
            pltpu.SemaphoreType.DMA((N_DEV,)),
            pltpu.SemaphoreType.DMA((N_DEV,)),
        ],
        compiler_params=pltpu.CompilerParams(collective_id=0),
    )(x, router_W, route_idx, expert_W)


# device time: 3246 ns/iter; 3.3728x vs baseline; 2.9732x over previous
import jax
import jax.numpy as jnp
from jax import lax
from jax.experimental import pallas as pl
from jax.experimental.pallas import tpu as pltpu

N_DEV = 8
N_TOK = 256
D_IN = 128
D_OUT = 256
N_EXP = 16
EXP_PER_DEV = 2
ROWS = N_TOK // N_DEV


def kernel(x, router_W, route_idx, expert_W):
    def body(x_ref, rw_ref, idx_ref, ew_ref, out_ref,
             partial_ref, acc_ref, send_sems, recv_sems, credit_sems):
        my = lax.axis_index("i")

        bar = pltpu.get_barrier_semaphore()
        pl.semaphore_signal(bar, inc=1)
        pl.semaphore_wait(bar, 1)

        for k in range(1, N_DEV):
            s = lax.rem(my - k + N_DEV, N_DEV)
            pl.semaphore_signal(
                credit_sems.at[k], inc=1, device_id=(s,),
                device_id_type=pl.DeviceIdType.MESH,
            )

        xf = x_ref[:, :]
        scores = jnp.dot(xf, rw_ref[:, :], preferred_element_type=jnp.float32)
        smax = jnp.max(scores, axis=1, keepdims=True)
        es = jnp.exp(scores - smax)
        eidx = lax.broadcasted_iota(jnp.int32, (N_TOK, N_EXP), 1)
        i0 = idx_ref[:, 0:1]
        i1 = idx_ref[:, 1:2]
        p0 = jnp.sum(jnp.where(eidx == i0, es, 0.0), axis=1, keepdims=True)
        p1 = jnp.sum(jnp.where(eidx == i1, es, 0.0), axis=1, keepdims=True)
        gs = p0 + p1

        xb = xf.astype(jnp.bfloat16)
        partial = jnp.zeros((N_TOK, D_OUT), jnp.float32)
        for le in range(EXP_PER_DEV):
            eg = my * EXP_PER_DEV + le
            w = (jnp.where(i0 == eg, p0, 0.0)
                 + jnp.where(i1 == eg, p1, 0.0)) / gs
            y = jnp.dot(xb, ew_ref[le].astype(jnp.bfloat16),
                        preferred_element_type=jnp.float32)
            partial = partial + w * y
        partial_ref[:, :] = partial.astype(jnp.bfloat16)

        rdmas = []
        for k in range(1, N_DEV):
            t = lax.rem(my + k, N_DEV)
            pl.semaphore_wait(credit_sems.at[k], 1)
            rdma = pltpu.make_async_remote_copy(
                src_ref=partial_ref.at[pl.ds(t * ROWS, ROWS)],
                dst_ref=acc_ref.at[k],
                send_sem=send_sems.at[k],
                recv_sem=recv_sems.at[k],
                device_id=(t,),
                device_id_type=pl.DeviceIdType.MESH,
            )
            rdma.start()
            rdmas.append(rdma)

        out = partial_ref[pl.ds(my * ROWS, ROWS), :].astype(jnp.float32)
        for k, rdma in zip(range(1, N_DEV), rdmas):
            rdma.wait_recv()
            out = out + acc_ref[k].astype(jnp.float32)
        out_ref[:, :] = out
        for rdma in rdmas:
            rdma.wait_send()

    return pl.pallas_call(
        body,
        out_shape=jax.ShapeDtypeStruct((ROWS, D_OUT), jnp.float32),
        in_specs=[pl.BlockSpec(memory_space=pltpu.VMEM)] * 4,
        out_specs=pl.BlockSpec(memory_space=pltpu.VMEM),
        scratch_shapes=[
            pltpu.VMEM((N_TOK, D_OUT), jnp.bfloat16),
            pltpu.VMEM((N_DEV, ROWS, D_OUT), jnp.bfloat16),
            pltpu.SemaphoreType.DMA((N_DEV,)),
            pltpu.SemaphoreType.DMA((N_DEV,)),
            pltpu.SemaphoreType.REGULAR((N_DEV,)),
        ],
        compiler_params=pltpu.CompilerParams(collective_id=0),
    )(x, router_W, route_idx, expert_W)
